# baseline (speedup 1.0000x reference)
import functools

import jax
import jax.numpy as jnp
import numpy as np
from jax import lax
from jax.experimental import pallas as pl
from jax.experimental.pallas import tpu as pltpu
from jax.experimental.pallas import tpu_sc as plsc

L = 16
T = 524288
F = 2
BASE_RES = 16
PER_LEVEL_SCALE = 1.3819128800392342
N = 262144
P1 = np.uint32(2654435761)
P2 = np.uint32(805459861)
RES = [int(np.floor(BASE_RES * PER_LEVEL_SCALE ** l)) for l in range(L)]

NC, NS, LANES = 2, 16, 16
NW = NC * NS
PTS_W = N // NW
C = 512
NCHUNK = PTS_W // C
NGRP = C // LANES
R = 8 * C

NGL = 2
GR1 = [RES[l] + 1 for l in range(NGL)]
GVG = [r * r * r for r in GR1]
GPAD = [(v + R - 1) // R * R for v in GVG]
GOFF = [sum(GPAD[:l]) for l in range(NGL)]
GTOT = sum(GPAD)

CORNERS = [(dx, dy, dz) for dx in (0, 1) for dy in (0, 1) for dz in (0, 1)]


def _compute_level(l, xyz_v, idx_ref, w_ref):
    res = jnp.float32(RES[l])
    lofs = jnp.int32(l * T)
    mask = jnp.uint32(T - 1)

    def g_body(g, carry):
        s = g * LANES
        x = xyz_v[0][pl.ds(s, LANES)]
        y = xyz_v[1][pl.ds(s, LANES)]
        z = xyz_v[2][pl.ds(s, LANES)]
        xs = x * res
        ys = y * res
        zs = z * res
        xi = xs.astype(jnp.int32)
        yi = ys.astype(jnp.int32)
        zi = zs.astype(jnp.int32)
        wx = xs - xi.astype(jnp.float32)
        wy = ys - yi.astype(jnp.float32)
        wz = zs - zi.astype(jnp.float32)
        hx0 = xi.astype(jnp.uint32)
        hy0 = yi.astype(jnp.uint32) * P1
        hz0 = zi.astype(jnp.uint32) * P2
        hx1 = hx0 + jnp.uint32(1)
        hy1 = hy0 + P1
        hz1 = hz0 + P2
        ux = 1.0 - wx
        uy = 1.0 - wy
        uz = 1.0 - wz
        a00 = ux * uy
        a01 = ux * wy
        a10 = wx * uy
        a11 = wx * wy
        wxy = {(0, 0): a00, (0, 1): a01, (1, 0): a10, (1, 1): a11}
        for c, (dx, dy, dz) in enumerate(CORNERS):
            hx = hx1 if dx else hx0
            hy = hy1 if dy else hy0
            hz = hz1 if dz else hz0
            h = (((hx ^ hy ^ hz) & mask)).astype(jnp.int32) + lofs
            idx_ref[pl.ds(c * C + s, LANES)] = h
            w_ref[pl.ds(c * C + s, LANES)] = wxy[(dx, dy)] * (wz if dz else uz)
        return carry

    lax.fori_loop(0, NGRP, g_body, 0, unroll=False)


def _build_grids(tab0_hbm, tab1_hbm, idx_ref, grid0, grid1, sem_a, sem_b):
    lane = lax.iota(jnp.int32, LANES)
    mask = jnp.uint32(T - 1)
    for l in range(NGL):
        r1 = GR1[l]
        vg = GVG[l]
        lofs = jnp.int32(l * T)
        for p0 in range(0, GPAD[l], R):
            def g_body(g, carry):
                v = p0 + g * LANES + lane
                vc = jnp.minimum(v, vg - 1)
                i = vc // (r1 * r1)
                rem = vc - i * (r1 * r1)
                j = rem // r1
                k = rem - j * r1
                h = (i.astype(jnp.uint32)
                     ^ (j.astype(jnp.uint32) * P1)
                     ^ (k.astype(jnp.uint32) * P2)) & mask
                idx_ref[pl.ds(g * LANES, LANES)] = h.astype(jnp.int32) + lofs
                return carry

            lax.fori_loop(0, R // LANES, g_body, 0, unroll=False)
            d0 = pltpu.async_copy(tab0_hbm.at[idx_ref],
                                  grid0.at[pl.ds(GOFF[l] + p0, R)], sem_a)
            d1 = pltpu.async_copy(tab1_hbm.at[idx_ref],
                                  grid1.at[pl.ds(GOFF[l] + p0, R)], sem_b)
            d0.wait()
            d1.wait()


def _grid_level(l, xyz_v, grid0, grid1, enc_v):
    res = jnp.float32(RES[l])
    r1 = GR1[l]
    goff = GOFF[l]

    def g_body(g, carry):
        s = g * LANES
        x = xyz_v[0][pl.ds(s, LANES)]
        y = xyz_v[1][pl.ds(s, LANES)]
        z = xyz_v[2][pl.ds(s, LANES)]
        xs = x * res
        ys = y * res
        zs = z * res
        xi = xs.astype(jnp.int32)
        yi = ys.astype(jnp.int32)
        zi = zs.astype(jnp.int32)
        wx = xs - xi.astype(jnp.float32)
        wy = ys - yi.astype(jnp.float32)
        wz = zs - zi.astype(jnp.float32)
        gbase = (xi * r1 + yi) * r1 + zi + goff
        ux = 1.0 - wx
        uy = 1.0 - wy
        uz = 1.0 - wz
        a00 = ux * uy
        a01 = ux * wy
        a10 = wx * uy
        a11 = wx * wy
        wxy = {(0, 0): a00, (0, 1): a01, (1, 0): a10, (1, 1): a11}
        acc_a = None
        for c, (dx, dy, dz) in enumerate(CORNERS):
            gi = gbase + (dx * r1 * r1 + dy * r1 + dz)
            fa = plsc.load_gather(grid0, [gi])
            fb = plsc.load_gather(grid1, [gi])
            wv = wxy[(dx, dy)] * (wz if dz else uz)
            if acc_a is None:
                acc_a = wv * fa
                acc_b = wv * fb
            else:
                acc_a = acc_a + wv * fa
                acc_b = acc_b + wv * fb
        enc_v[2 * l, pl.ds(s, LANES)] = acc_a
        enc_v[2 * l + 1, pl.ds(s, LANES)] = acc_b
        return carry

    lax.fori_loop(0, NGRP, g_body, 0, unroll=False)


def _accum_level(l, rows0_ref, rows1_ref, w_ref, enc_v):
    def g_body(g, carry):
        s = g * LANES
        acc_a = None
        for c in range(8):
            r0 = c * C + s
            fa = rows0_ref[pl.ds(r0, LANES)]
            fb = rows1_ref[pl.ds(r0, LANES)]
            wv = w_ref[pl.ds(r0, LANES)]
            if acc_a is None:
                acc_a = wv * fa
                acc_b = wv * fb
            else:
                acc_a = acc_a + wv * fa
                acc_b = acc_b + wv * fb
        enc_v[2 * l, pl.ds(s, LANES)] = acc_a
        enc_v[2 * l + 1, pl.ds(s, LANES)] = acc_b
        return carry

    lax.fori_loop(0, NGRP, g_body, 0, unroll=False)


def _enc_body(x_hbm, y_hbm, z_hbm, tab0_hbm, tab1_hbm, enc_hbm,
              xv, yv, zv, idx_a, idx_b,
              rows_a0, rows_a1, rows_b0, rows_b1, w_a, w_b, enc_v,
              grid0, grid1, sem_a, sem_b):
    cid = lax.axis_index("c")
    sid = lax.axis_index("s")
    wid = sid * NC + cid
    base = wid * PTS_W

    xyz_v = (xv, yv, zv)
    idx_bufs = (idx_a, idx_b)
    rows_bufs = ((rows_a0, rows_a1), (rows_b0, rows_b1))
    w_bufs = (w_a, w_b)
    sems = (sem_a, sem_b)

    _build_grids(tab0_hbm, tab1_hbm, idx_a, grid0, grid1, sem_a, sem_b)

    def fire(slot):
        d0 = pltpu.async_copy(tab0_hbm.at[idx_bufs[slot]],
                              rows_bufs[slot][0], sems[slot])
        d1 = pltpu.async_copy(tab1_hbm.at[idx_bufs[slot]],
                              rows_bufs[slot][1], sems[slot])
        return (d0, d1)

    def chunk_body(ci, carry):
        off = base + ci * C
        pltpu.sync_copy(x_hbm.at[pl.ds(off, C)], xv)
        pltpu.sync_copy(y_hbm.at[pl.ds(off, C)], yv)
        pltpu.sync_copy(z_hbm.at[pl.ds(off, C)], zv)
        _compute_level(NGL, xyz_v, idx_bufs[0], w_bufs[0])
        descs = [None, None]
        descs[0] = fire(0)
        for l in range(NGL):
            _grid_level(l, xyz_v, grid0, grid1, enc_v)
        for l in range(NGL, L):
            cur = (l - NGL) % 2
            nxt = (l - NGL + 1) % 2
            if l + 1 < L:
                _compute_level(l + 1, xyz_v, idx_bufs[nxt], w_bufs[nxt])
                descs[nxt] = fire(nxt)
            descs[cur][0].wait()
            descs[cur][1].wait()
            _accum_level(l, rows_bufs[cur][0], rows_bufs[cur][1],
                         w_bufs[cur], enc_v)
        pltpu.sync_copy(enc_v, enc_hbm.at[:, pl.ds(off, C)])
        return carry

    lax.fori_loop(0, NCHUNK, chunk_body, 0, unroll=False)


@functools.partial(
    pl.kernel,
    out_type=jax.ShapeDtypeStruct((L * F, N), jnp.float32),
    mesh=plsc.VectorSubcoreMesh(core_axis_name="c", subcore_axis_name="s"),
    scratch_types=[
        pltpu.VMEM((C,), jnp.float32),
        pltpu.VMEM((C,), jnp.float32),
        pltpu.VMEM((C,), jnp.float32),
        pltpu.VMEM((R,), jnp.int32),
        pltpu.VMEM((R,), jnp.int32),
        pltpu.VMEM((R,), jnp.float32),
        pltpu.VMEM((R,), jnp.float32),
        pltpu.VMEM((R,), jnp.float32),
        pltpu.VMEM((R,), jnp.float32),
        pltpu.VMEM((R,), jnp.float32),
        pltpu.VMEM((R,), jnp.float32),
        pltpu.VMEM((L * F, C), jnp.float32),
        pltpu.VMEM((GTOT,), jnp.float32),
        pltpu.VMEM((GTOT,), jnp.float32),
        pltpu.SemaphoreType.DMA,
        pltpu.SemaphoreType.DMA,
    ],
    compiler_params=pltpu.CompilerParams(needs_layout_passes=False),
)
def _encode_sc(x_hbm, y_hbm, z_hbm, tab0_hbm, tab1_hbm, enc_hbm, *rest):
    _enc_body(x_hbm, y_hbm, z_hbm, tab0_hbm, tab1_hbm, enc_hbm, *rest)


BLK = 8192


def _mlp_body(enc_ref, w0_ref, w1_ref, w2_ref, out_ref):
    h = lax.dot_general(enc_ref[...], w0_ref[...], (((0,), (0,)), ((), ())),
                        preferred_element_type=jnp.float32)
    h = jnp.maximum(h, 0.0)
    h = jnp.dot(h, w1_ref[...], preferred_element_type=jnp.float32)
    h = jnp.maximum(h, 0.0)
    out_ref[...] = jnp.dot(h, w2_ref[...], preferred_element_type=jnp.float32)


def _mlp(enc, W0, W1, W2):
    return pl.pallas_call(
        _mlp_body,
        grid=(N // BLK,),
        in_specs=[
            pl.BlockSpec((L * F, BLK), lambda i: (0, i)),
            pl.BlockSpec((L * F, 64), lambda i: (0, 0)),
            pl.BlockSpec((64, 64), lambda i: (0, 0)),
            pl.BlockSpec((64, 1), lambda i: (0, 0)),
        ],
        out_specs=pl.BlockSpec((BLK, 1), lambda i: (i, 0)),
        out_shape=jax.ShapeDtypeStruct((N, 1), jnp.float32),
    )(enc, W0, W1, W2)


def kernel(inputs, table, W0, W1, W2):
    x1 = inputs[:, 0]
    y1 = inputs[:, 1]
    z1 = inputs[:, 2]
    tab0 = table[:, :, 0].reshape(L * T)
    tab1 = table[:, :, 1].reshape(L * T)
    enc = _encode_sc(x1, y1, z1, tab0, tab1)
    return _mlp(enc, W0, W1, W2)


# speedup vs baseline: 1.0657x; 1.0657x over previous
import functools

import jax
import jax.numpy as jnp
import numpy as np
from jax import lax
from jax.experimental import pallas as pl
from jax.experimental.pallas import tpu as pltpu
from jax.experimental.pallas import tpu_sc as plsc

L = 16
T = 524288
F = 2
BASE_RES = 16
PER_LEVEL_SCALE = 1.3819128800392342
N = 262144
P1 = np.uint32(2654435761)
P2 = np.uint32(805459861)
RES = [int(np.floor(BASE_RES * PER_LEVEL_SCALE ** l)) for l in range(L)]

NC, NS, LANES = 2, 16, 16
NW = NC * NS
PTS_W = N // NW
C = 1024
NCHUNK = PTS_W // C
NGRP = C // LANES
R = 8 * C

CORNERS = [(dx, dy, dz) for dx in (0, 1) for dy in (0, 1) for dz in (0, 1)]


def _compute_level(l, xyz_v, idx_ref, w_ref):
    res = jnp.float32(RES[l])
    lofs = jnp.int32(l * T)
    mask = jnp.uint32(T - 1)

    def g_body(g, carry):
        s = g * LANES
        x = xyz_v[0][pl.ds(s, LANES)]
        y = xyz_v[1][pl.ds(s, LANES)]
        z = xyz_v[2][pl.ds(s, LANES)]
        xs = x * res
        ys = y * res
        zs = z * res
        xi = xs.astype(jnp.int32)
        yi = ys.astype(jnp.int32)
        zi = zs.astype(jnp.int32)
        wx = xs - xi.astype(jnp.float32)
        wy = ys - yi.astype(jnp.float32)
        wz = zs - zi.astype(jnp.float32)
        hx0 = xi.astype(jnp.uint32)
        hy0 = yi.astype(jnp.uint32) * P1
        hz0 = zi.astype(jnp.uint32) * P2
        hx1 = hx0 + jnp.uint32(1)
        hy1 = hy0 + P1
        hz1 = hz0 + P2
        ux = 1.0 - wx
        uy = 1.0 - wy
        uz = 1.0 - wz
        a00 = ux * uy
        a01 = ux * wy
        a10 = wx * uy
        a11 = wx * wy
        wxy = {(0, 0): a00, (0, 1): a01, (1, 0): a10, (1, 1): a11}
        for c, (dx, dy, dz) in enumerate(CORNERS):
            hx = hx1 if dx else hx0
            hy = hy1 if dy else hy0
            hz = hz1 if dz else hz0
            h = (((hx ^ hy ^ hz) & mask)).astype(jnp.int32) + lofs
            idx_ref[pl.ds(c * C + s, LANES)] = h
            w_ref[pl.ds(c * C + s, LANES)] = wxy[(dx, dy)] * (wz if dz else uz)
        return carry

    lax.fori_loop(0, NGRP, g_body, 0, unroll=False)


def _accum_level(l, rows0_ref, rows1_ref, w_ref, enc_v):
    def g_body(g, carry):
        s = g * LANES
        acc_a = None
        for c in range(8):
            r0 = c * C + s
            fa = rows0_ref[pl.ds(r0, LANES)]
            fb = rows1_ref[pl.ds(r0, LANES)]
            wv = w_ref[pl.ds(r0, LANES)]
            if acc_a is None:
                acc_a = wv * fa
                acc_b = wv * fb
            else:
                acc_a = acc_a + wv * fa
                acc_b = acc_b + wv * fb
        enc_v[2 * l, pl.ds(s, LANES)] = acc_a
        enc_v[2 * l + 1, pl.ds(s, LANES)] = acc_b
        return carry

    lax.fori_loop(0, NGRP, g_body, 0, unroll=False)


def _enc_body(x_hbm, y_hbm, z_hbm, tab0_hbm, tab1_hbm, enc_hbm,
              xv, yv, zv, idx_a, idx_b,
              rows_a0, rows_a1, rows_b0, rows_b1, w_a, w_b, enc_v,
              sem_a, sem_b):
    cid = lax.axis_index("c")
    sid = lax.axis_index("s")
    wid = sid * NC + cid
    base = wid * PTS_W

    xyz_v = (xv, yv, zv)
    idx_bufs = (idx_a, idx_b)
    rows_bufs = ((rows_a0, rows_a1), (rows_b0, rows_b1))
    w_bufs = (w_a, w_b)
    sems = (sem_a, sem_b)

    def fire(slot):
        d0 = pltpu.async_copy(tab0_hbm.at[idx_bufs[slot]],
                              rows_bufs[slot][0], sems[slot])
        d1 = pltpu.async_copy(tab1_hbm.at[idx_bufs[slot]],
                              rows_bufs[slot][1], sems[slot])
        return (d0, d1)

    def chunk_body(ci, carry):
        off = base + ci * C
        pltpu.sync_copy(x_hbm.at[pl.ds(off, C)], xv)
        pltpu.sync_copy(y_hbm.at[pl.ds(off, C)], yv)
        pltpu.sync_copy(z_hbm.at[pl.ds(off, C)], zv)
        _compute_level(0, xyz_v, idx_bufs[0], w_bufs[0])
        descs = [None, None]
        descs[0] = fire(0)
        for l in range(L):
            cur = l % 2
            nxt = (l + 1) % 2
            if l + 1 < L:
                _compute_level(l + 1, xyz_v, idx_bufs[nxt], w_bufs[nxt])
                descs[nxt] = fire(nxt)
            descs[cur][0].wait()
            descs[cur][1].wait()
            _accum_level(l, rows_bufs[cur][0], rows_bufs[cur][1],
                         w_bufs[cur], enc_v)
        pltpu.sync_copy(enc_v, enc_hbm.at[:, pl.ds(off, C)])
        return carry

    lax.fori_loop(0, NCHUNK, chunk_body, 0, unroll=False)


@functools.partial(
    pl.kernel,
    out_type=jax.ShapeDtypeStruct((L * F, N), jnp.float32),
    mesh=plsc.VectorSubcoreMesh(core_axis_name="c", subcore_axis_name="s"),
    scratch_types=[
        pltpu.VMEM((C,), jnp.float32),
        pltpu.VMEM((C,), jnp.float32),
        pltpu.VMEM((C,), jnp.float32),
        pltpu.VMEM((R,), jnp.int32),
        pltpu.VMEM((R,), jnp.int32),
        pltpu.VMEM((R,), jnp.float32),
        pltpu.VMEM((R,), jnp.float32),
        pltpu.VMEM((R,), jnp.float32),
        pltpu.VMEM((R,), jnp.float32),
        pltpu.VMEM((R,), jnp.float32),
        pltpu.VMEM((R,), jnp.float32),
        pltpu.VMEM((L * F, C), jnp.float32),
        pltpu.SemaphoreType.DMA,
        pltpu.SemaphoreType.DMA,
    ],
)
def _encode_sc(x_hbm, y_hbm, z_hbm, tab0_hbm, tab1_hbm, enc_hbm, *rest):
    _enc_body(x_hbm, y_hbm, z_hbm, tab0_hbm, tab1_hbm, enc_hbm, *rest)


BLK = 8192


def _mlp_body(enc_ref, w0_ref, w1_ref, w2_ref, out_ref):
    h = lax.dot_general(enc_ref[...], w0_ref[...], (((0,), (0,)), ((), ())),
                        preferred_element_type=jnp.float32)
    h = jnp.maximum(h, 0.0)
    h = jnp.dot(h, w1_ref[...], preferred_element_type=jnp.float32)
    h = jnp.maximum(h, 0.0)
    out_ref[...] = jnp.dot(h, w2_ref[...], preferred_element_type=jnp.float32)


def _mlp(enc, W0, W1, W2):
    return pl.pallas_call(
        _mlp_body,
        grid=(N // BLK,),
        in_specs=[
            pl.BlockSpec((L * F, BLK), lambda i: (0, i)),
            pl.BlockSpec((L * F, 64), lambda i: (0, 0)),
            pl.BlockSpec((64, 64), lambda i: (0, 0)),
            pl.BlockSpec((64, 1), lambda i: (0, 0)),
        ],
        out_specs=pl.BlockSpec((BLK, 1), lambda i: (i, 0)),
        out_shape=jax.ShapeDtypeStruct((N, 1), jnp.float32),
    )(enc, W0, W1, W2)


def kernel(inputs, table, W0, W1, W2):
    x1 = inputs[:, 0]
    y1 = inputs[:, 1]
    z1 = inputs[:, 2]
    tab0 = table[:, :, 0].reshape(L * T)
    tab1 = table[:, :, 1].reshape(L * T)
    enc = _encode_sc(x1, y1, z1, tab0, tab1)
    return _mlp(enc, W0, W1, W2)


# speedup vs baseline: 1.0674x; 1.0016x over previous
import functools

import jax
import jax.numpy as jnp
import numpy as np
from jax import lax
from jax.experimental import pallas as pl
from jax.experimental.pallas import tpu as pltpu
from jax.experimental.pallas import tpu_sc as plsc

L = 16
T = 524288
F = 2
BASE_RES = 16
PER_LEVEL_SCALE = 1.3819128800392342
N = 262144
P1 = np.uint32(2654435761)
P2 = np.uint32(805459861)
RES = [int(np.floor(BASE_RES * PER_LEVEL_SCALE ** l)) for l in range(L)]

NC, NS, LANES = 2, 16, 16
NW = NC * NS
PTS_W = N // NW
C = 1024
NCHUNK = PTS_W // C
NGRP = C // LANES
R = 8 * C

CORNERS = [(dx, dy, dz) for dx in (0, 1) for dy in (0, 1) for dz in (0, 1)]


def _compute_level(l, xyz_v, idx_ref, w_ref):
    res = jnp.float32(RES[l])
    lofs = jnp.int32(l * T)
    mask = jnp.uint32(T - 1)

    def g_body(g, carry):
        s = g * LANES
        x = xyz_v[0][pl.ds(s, LANES)]
        y = xyz_v[1][pl.ds(s, LANES)]
        z = xyz_v[2][pl.ds(s, LANES)]
        xs = x * res
        ys = y * res
        zs = z * res
        xi = xs.astype(jnp.int32)
        yi = ys.astype(jnp.int32)
        zi = zs.astype(jnp.int32)
        wx = xs - xi.astype(jnp.float32)
        wy = ys - yi.astype(jnp.float32)
        wz = zs - zi.astype(jnp.float32)
        hx0 = xi.astype(jnp.uint32)
        hy0 = yi.astype(jnp.uint32) * P1
        hz0 = zi.astype(jnp.uint32) * P2
        hx1 = hx0 + jnp.uint32(1)
        hy1 = hy0 + P1
        hz1 = hz0 + P2
        ux = 1.0 - wx
        uy = 1.0 - wy
        uz = 1.0 - wz
        a00 = ux * uy
        a01 = ux * wy
        a10 = wx * uy
        a11 = wx * wy
        wxy = {(0, 0): a00, (0, 1): a01, (1, 0): a10, (1, 1): a11}
        for c, (dx, dy, dz) in enumerate(CORNERS):
            hx = hx1 if dx else hx0
            hy = hy1 if dy else hy0
            hz = hz1 if dz else hz0
            h = (((hx ^ hy ^ hz) & mask)).astype(jnp.int32) + lofs
            idx_ref[pl.ds(c * C + s, LANES)] = h
            w_ref[pl.ds(c * C + s, LANES)] = wxy[(dx, dy)] * (wz if dz else uz)
        return carry

    lax.fori_loop(0, NGRP, g_body, 0, unroll=False)


def _accum_level(l, rows0_ref, rows1_ref, w_ref, enc_v):
    def g_body(g, carry):
        s = g * LANES
        acc_a = None
        for c in range(8):
            r0 = c * C + s
            fa = rows0_ref[pl.ds(r0, LANES)]
            fb = rows1_ref[pl.ds(r0, LANES)]
            wv = w_ref[pl.ds(r0, LANES)]
            if acc_a is None:
                acc_a = wv * fa
                acc_b = wv * fb
            else:
                acc_a = acc_a + wv * fa
                acc_b = acc_b + wv * fb
        enc_v[2 * l, pl.ds(s, LANES)] = acc_a
        enc_v[2 * l + 1, pl.ds(s, LANES)] = acc_b
        return carry

    lax.fori_loop(0, NGRP, g_body, 0, unroll=False)


def _enc_body(x_hbm, y_hbm, z_hbm, tab0_hbm, tab1_hbm, enc_hbm,
              xv, yv, zv, idx_a, idx_b,
              rows_a0, rows_a1, rows_b0, rows_b1, w_a, w_b, enc_v,
              sem_a, sem_b):
    cid = lax.axis_index("c")
    sid = lax.axis_index("s")
    wid = sid * NC + cid
    base = wid * PTS_W

    xyz_v = (xv, yv, zv)
    idx_bufs = (idx_a, idx_b)
    rows_bufs = ((rows_a0, rows_a1), (rows_b0, rows_b1))
    w_bufs = (w_a, w_b)
    sems = (sem_a, sem_b)

    def fire(slot):
        pltpu.async_copy(tab0_hbm.at[idx_bufs[slot]],
                         rows_bufs[slot][0], sems[slot])
        pltpu.async_copy(tab1_hbm.at[idx_bufs[slot]],
                         rows_bufs[slot][1], sems[slot])

    def wait(slot):
        pltpu.make_async_copy(tab0_hbm.at[idx_bufs[slot]],
                              rows_bufs[slot][0], sems[slot]).wait()
        pltpu.make_async_copy(tab1_hbm.at[idx_bufs[slot]],
                              rows_bufs[slot][1], sems[slot]).wait()

    def start_chunk(off):
        pltpu.sync_copy(x_hbm.at[pl.ds(off, C)], xv)
        pltpu.sync_copy(y_hbm.at[pl.ds(off, C)], yv)
        pltpu.sync_copy(z_hbm.at[pl.ds(off, C)], zv)
        _compute_level(0, xyz_v, idx_bufs[0], w_bufs[0])
        fire(0)

    start_chunk(base)

    def chunk_body(ci, carry):
        off = base + ci * C
        for l in range(L):
            cur = l % 2
            nxt = (l + 1) % 2
            if l + 1 < L:
                _compute_level(l + 1, xyz_v, idx_bufs[nxt], w_bufs[nxt])
                fire(nxt)
            if l == L - 1:
                @pl.when(ci + 1 < NCHUNK)
                def _():
                    start_chunk(off + C)
            wait(cur)
            _accum_level(l, rows_bufs[cur][0], rows_bufs[cur][1],
                         w_bufs[cur], enc_v)
        pltpu.sync_copy(enc_v, enc_hbm.at[:, pl.ds(off, C)])
        return carry

    lax.fori_loop(0, NCHUNK, chunk_body, 0, unroll=False)


@functools.partial(
    pl.kernel,
    out_type=jax.ShapeDtypeStruct((L * F, N), jnp.float32),
    mesh=plsc.VectorSubcoreMesh(core_axis_name="c", subcore_axis_name="s"),
    scratch_types=[
        pltpu.VMEM((C,), jnp.float32),
        pltpu.VMEM((C,), jnp.float32),
        pltpu.VMEM((C,), jnp.float32),
        pltpu.VMEM((R,), jnp.int32),
        pltpu.VMEM((R,), jnp.int32),
        pltpu.VMEM((R,), jnp.float32),
        pltpu.VMEM((R,), jnp.float32),
        pltpu.VMEM((R,), jnp.float32),
        pltpu.VMEM((R,), jnp.float32),
        pltpu.VMEM((R,), jnp.float32),
        pltpu.VMEM((R,), jnp.float32),
        pltpu.VMEM((L * F, C), jnp.float32),
        pltpu.SemaphoreType.DMA,
        pltpu.SemaphoreType.DMA,
    ],
)
def _encode_sc(x_hbm, y_hbm, z_hbm, tab0_hbm, tab1_hbm, enc_hbm, *rest):
    _enc_body(x_hbm, y_hbm, z_hbm, tab0_hbm, tab1_hbm, enc_hbm, *rest)


BLK = 8192


def _mlp_body(enc_ref, w0_ref, w1_ref, w2_ref, out_ref):
    h = lax.dot_general(enc_ref[...], w0_ref[...], (((0,), (0,)), ((), ())),
                        preferred_element_type=jnp.float32)
    h = jnp.maximum(h, 0.0)
    h = jnp.dot(h, w1_ref[...], preferred_element_type=jnp.float32)
    h = jnp.maximum(h, 0.0)
    out_ref[...] = jnp.dot(h, w2_ref[...], preferred_element_type=jnp.float32)


def _mlp(enc, W0, W1, W2):
    return pl.pallas_call(
        _mlp_body,
        grid=(N // BLK,),
        in_specs=[
            pl.BlockSpec((L * F, BLK), lambda i: (0, i)),
            pl.BlockSpec((L * F, 64), lambda i: (0, 0)),
            pl.BlockSpec((64, 64), lambda i: (0, 0)),
            pl.BlockSpec((64, 1), lambda i: (0, 0)),
        ],
        out_specs=pl.BlockSpec((BLK, 1), lambda i: (i, 0)),
        out_shape=jax.ShapeDtypeStruct((N, 1), jnp.float32),
    )(enc, W0, W1, W2)


def kernel(inputs, table, W0, W1, W2):
    x1 = inputs[:, 0]
    y1 = inputs[:, 1]
    z1 = inputs[:, 2]
    tab0 = table[:, :, 0].reshape(L * T)
    tab1 = table[:, :, 1].reshape(L * T)
    enc = _encode_sc(x1, y1, z1, tab0, tab1)
    return _mlp(enc, W0, W1, W2)
